# baseline (speedup 1.0000x reference)
import functools

import jax
import jax.numpy as jnp
from jax.experimental import pallas as pl

_B, _S, _D = 2, 2048, 768
_G, _FPG, _DFE, _DH, _DRH = 8, 8, 64, 64, 64
_GH = _G * _DH


def _gelu(x):
    return 0.5 * x * (1.0 + jax.lax.erf(x * 0.7071067811865476))


def _moe_body(x_ref, f_ref, pk_ref, out_ref):
    wh = pk_ref[0:_D, :]
    wgbd = pk_ref[_D:_D + 64, 0:_GH]
    wr1e = pk_ref[832:1344, 0:_GH]
    wr2bd = pk_ref[832:1344, _GH:_GH + _G]
    we2 = pk_ref[1344:1856, 0:_D]
    be2 = pk_ref[1856:1864, 0:_D]
    lng = pk_ref[1864:1865, 0:_D]
    lnb = pk_ref[1865:1866, 0:_D]
    bgf = pk_ref[1866:1867, 0:_GH]
    br1f = pk_ref[1866:1867, _GH:2 * _GH]
    be1f = pk_ref[1867:1868, 0:_GH]
    br2f = pk_ref[1867:1868, _GH:_GH + _G]

    x = x_ref[...]
    mu = jnp.mean(x, axis=1, keepdims=True)
    xc = x - mu
    var = jnp.mean(xc * xc, axis=1, keepdims=True)
    h = xc * jax.lax.rsqrt(var + 1e-5) * lng + lnb

    dot = functools.partial(jnp.dot, preferred_element_type=jnp.float32)
    hw = dot(h, wh)
    emb = dot(f_ref[...], wgbd) + bgf
    r1 = _gelu(hw[:, :_GH] + dot(emb, wr1e) + br1f)
    e1 = _gelu(hw[:, _GH:] + be1f)

    logits = dot(r1, wr2bd) + br2f
    m1 = jnp.max(logits, axis=1, keepdims=True)
    l2 = jnp.where(logits == m1, -jnp.inf, logits)
    m2 = jnp.max(l2, axis=1, keepdims=True)
    inv = 1.0 / (1.0 + jnp.exp(m2 - m1))
    gw = jnp.where(logits >= m2, jnp.exp(logits - m1), 0.0) * inv

    r8 = jax.lax.broadcasted_iota(jnp.int32, (_G, _GH), 0)
    c512 = jax.lax.broadcasted_iota(jnp.int32, (_G, _GH), 1)
    spread = (c512 // _DH == r8).astype(jnp.float32)

    e1w = e1 * dot(gw, spread)
    out_ref[...] = dot(e1w, we2) + dot(gw, be2)


def kernel(hidden, features, ln_g, ln_b, Wg, bg, Wr1, br1, Wr2, br2,
           We1, be1, We2, be2):
    n = _B * _S
    x2 = hidden.reshape(n, _D)
    f2 = features.reshape(n, _G * _FPG)

    eye = jnp.eye(_G, dtype=jnp.float32)
    wg_bd = (eye[:, None, :, None] * Wg[:, :, None, :]).reshape(
        _G * _FPG, _G * _DFE)
    wr1e = (eye[:, None, :, None] * Wr1[:, _D:, :][:, :, None, :]).reshape(
        _G * _DFE, _G * _DRH)
    wr1h = Wr1[:, :_D, :].transpose(1, 0, 2).reshape(_D, _G * _DRH)
    we1c = We1.transpose(1, 0, 2).reshape(_D, _GH)
    wr2_bd = (eye[:, None, :] * Wr2[:, :, 0][:, :, None]).reshape(_GH, _G)
    we2c = We2.reshape(_GH, _D)

    z = lambda c: jnp.zeros((1, c), dtype=jnp.float32)
    packed = jnp.concatenate([
        jnp.concatenate([wr1h, we1c], axis=1),
        jnp.pad(wg_bd, ((0, 0), (0, _GH))),
        jnp.concatenate(
            [wr1e, wr2_bd, jnp.zeros((_GH, 1024 - _GH - _G))], axis=1),
        jnp.pad(we2c, ((0, 0), (0, 1024 - _D))),
        jnp.pad(be2, ((0, 0), (0, 1024 - _D))),
        jnp.pad(ln_g.reshape(1, _D), ((0, 0), (0, 1024 - _D))),
        jnp.pad(ln_b.reshape(1, _D), ((0, 0), (0, 1024 - _D))),
        jnp.concatenate(
            [bg.reshape(1, -1), br1.reshape(1, -1)], axis=1),
        jnp.concatenate(
            [be1.reshape(1, _GH), br2.reshape(1, _G), z(1024 - _GH - _G)],
            axis=1),
        jnp.zeros((4, 1024), dtype=jnp.float32),
    ], axis=0)

    tblk = 512
    grid = (n // tblk,)
    row = lambda i: (i, 0)

    out = pl.pallas_call(
        _moe_body,
        grid=grid,
        in_specs=[
            pl.BlockSpec((tblk, _D), row),
            pl.BlockSpec((tblk, _G * _FPG), row),
            pl.BlockSpec(packed.shape, lambda i: (0, 0)),
        ],
        out_specs=pl.BlockSpec((tblk, _D), row),
        out_shape=jax.ShapeDtypeStruct((n, _D), jnp.float32),
    )(x2, f2, packed)
    return out.reshape(_B, _S, _D)
